# baseline (speedup 1.0000x reference)
import functools

import jax
import jax.numpy as jnp
from jax import lax
from jax.experimental import pallas as pl
from jax.experimental.pallas import tpu as pltpu
from jax.experimental.pallas import tpu_sc as plsc

_B = 1024
_C = 256
_S = 4096
_NW = 32
_ROWS = _B // _NW
_ELEMS = _ROWS * _C
_L = 16


def _sc_gather(flat_maps, bids, finds):
    mesh = plsc.VectorSubcoreMesh(core_axis_name="c", subcore_axis_name="s")

    @functools.partial(
        pl.kernel,
        out_type=jax.ShapeDtypeStruct((_B * _C,), jnp.float32),
        mesh=mesh,
        scratch_types=[
            pltpu.VMEM((_ROWS,), jnp.int32),
            pltpu.VMEM((_ROWS,), jnp.int32),
            pltpu.VMEM((_ROWS + _L,), jnp.int32),
            pltpu.VMEM((_ELEMS,), jnp.int32),
            pltpu.VMEM((_ELEMS,), jnp.float32),
            pltpu.SemaphoreType.DMA,
        ],
    )
    def k(maps_hbm, bid_hbm, find_hbm, out_hbm,
          bid_v, find_v, base_v, idx_v, data_v, sem):
        wid = lax.axis_index("s") * 2 + lax.axis_index("c")
        row0 = wid * _ROWS
        pltpu.sync_copy(bid_hbm.at[pl.ds(row0, _ROWS)], bid_v)
        pltpu.sync_copy(find_hbm.at[pl.ds(row0, _ROWS)], find_v)
        for j in range(_ROWS // _L):
            sl = pl.ds(j * _L, _L)
            base_v[sl] = bid_v[sl] * (_C * _S) + find_v[sl]

        ramp = lax.iota(jnp.int32, _L) * _S

        def row_body(r, carry):
            b = base_v[pl.ds(r, _L)][0]
            for cc in range(_C // _L):
                idx_v[pl.ds(r * _C + cc * _L, _L)] = (
                    (ramp + cc * _L * _S) + b)
            return carry

        lax.fori_loop(0, _ROWS, row_body, 0)

        pltpu.async_copy(maps_hbm.at[idx_v], data_v, sem).wait()
        pltpu.sync_copy(data_v, out_hbm.at[pl.ds(wid * _ELEMS, _ELEMS)])

    return k(flat_maps, bids, finds)


def kernel(maps, batch_ids, flat_inds):
    batch, channel = maps.shape[0], maps.shape[1]
    flat = jnp.reshape(maps, (-1,))
    out = _sc_gather(flat,
                     batch_ids.astype(jnp.int32),
                     flat_inds.astype(jnp.int32))
    return jnp.reshape(out, (batch_ids.shape[0], channel))


# speedup vs baseline: 29.0250x; 29.0250x over previous
import functools

import jax
import jax.numpy as jnp
from jax import lax
from jax.experimental import pallas as pl
from jax.experimental.pallas import tpu as pltpu
from jax.experimental.pallas import tpu_sc as plsc

_B = 1024
_C = 256
_S = 4096
_NW = 32
_ROWS = _B // _NW
_L = 16


def _sc_gather(phys, bids, finds):
    mesh = plsc.VectorSubcoreMesh(core_axis_name="c", subcore_axis_name="s")

    @functools.partial(
        pl.kernel,
        out_type=jax.ShapeDtypeStruct((_B, _C), jnp.float32),
        mesh=mesh,
        scratch_types=[
            pltpu.VMEM((_ROWS,), jnp.int32),
            pltpu.VMEM((_ROWS,), jnp.int32),
            pltpu.VMEM((_ROWS,), jnp.int32),
            pltpu.VMEM((_ROWS, _C), jnp.float32),
            pltpu.SemaphoreType.DMA,
        ],
    )
    def k(phys_hbm, bid_hbm, find_hbm, out_hbm,
          bid_v, find_v, idx_v, rows_v, sem):
        wid = lax.axis_index("s") * 2 + lax.axis_index("c")
        row0 = wid * _ROWS
        pltpu.sync_copy(bid_hbm.at[pl.ds(row0, _ROWS)], bid_v)
        pltpu.sync_copy(find_hbm.at[pl.ds(row0, _ROWS)], find_v)
        for j in range(_ROWS // _L):
            sl = pl.ds(j * _L, _L)
            idx_v[sl] = bid_v[sl] * _S + find_v[sl]
        pltpu.async_copy(phys_hbm.at[idx_v], rows_v, sem).wait()
        pltpu.sync_copy(rows_v, out_hbm.at[pl.ds(row0, _ROWS)])

    return k(phys, bids, finds)


def kernel(maps, batch_ids, flat_inds):
    channel = maps.shape[1]
    phys = jnp.reshape(jnp.transpose(maps, (0, 2, 3, 1)), (-1, channel))
    return _sc_gather(phys,
                      batch_ids.astype(jnp.int32),
                      flat_inds.astype(jnp.int32))


# speedup vs baseline: 29.5485x; 1.0180x over previous
import functools

import jax
import jax.numpy as jnp
from jax import lax
from jax.experimental import pallas as pl
from jax.experimental.pallas import tpu as pltpu
from jax.experimental.pallas import tpu_sc as plsc

_B = 1024
_C = 256
_S = 4096
_NW = 32
_ROWS = _B // _NW
_L = 16


def _sc_gather(phys, bids, finds):
    mesh = plsc.VectorSubcoreMesh(core_axis_name="c", subcore_axis_name="s")

    @functools.partial(
        pl.kernel,
        out_type=jax.ShapeDtypeStruct((_B, _C), jnp.float32),
        mesh=mesh,
        scratch_types=[
            pltpu.VMEM((_ROWS,), jnp.int32),
            pltpu.VMEM((_ROWS,), jnp.int32),
            pltpu.VMEM((_ROWS,), jnp.int32),
            pltpu.VMEM((_ROWS, _C), jnp.float32),
            pltpu.SemaphoreType.DMA,
            pltpu.SemaphoreType.DMA,
        ],
    )
    def k(phys_hbm, bid_hbm, find_hbm, out_hbm,
          bid_v, find_v, idx_v, rows_v, sem1, sem2):
        wid = lax.axis_index("s") * 2 + lax.axis_index("c")
        row0 = wid * _ROWS
        cp1 = pltpu.async_copy(bid_hbm.at[pl.ds(row0, _ROWS)], bid_v, sem1)
        cp2 = pltpu.async_copy(find_hbm.at[pl.ds(row0, _ROWS)], find_v, sem2)
        cp1.wait()
        cp2.wait()
        for j in range(_ROWS // _L):
            sl = pl.ds(j * _L, _L)
            idx_v[sl] = bid_v[sl] * _S + find_v[sl]
        pltpu.async_copy(phys_hbm.at[idx_v], rows_v, sem1).wait()
        pltpu.sync_copy(rows_v, out_hbm.at[pl.ds(row0, _ROWS)])

    return k(phys, bids, finds)


def kernel(maps, batch_ids, flat_inds):
    channel = maps.shape[1]
    phys = jnp.reshape(jnp.transpose(maps, (0, 2, 3, 1)), (-1, channel))
    return _sc_gather(phys,
                      batch_ids.astype(jnp.int32),
                      flat_inds.astype(jnp.int32))
